# Initial kernel scaffold; baseline (speedup 1.0000x reference)
import jax
import jax.numpy as jnp
from jax.experimental import pallas as pl

_H1, _C1 = 8, 8


def _leaky(v):
    return jnp.where(v >= 0, v, 0.2 * v)


def _final_combine_kernel(num2_ref, den2_ref, g_ref, el_ref, b2_ref, out_ref):
    num2 = num2_ref[...]
    den2 = den2_ref[...]
    g = g_ref[...]
    el = el_ref[...]
    out_ref[...] = (num2 + el * g) / (den2 + el + 1e-16) + b2_ref[...]


def kernel(x, edge_index, edge_attr, W1, a1_src, a1_dst, We1, ae1, b1,
           W2, a2_src, a2_dst, We2, ae2, b2):
    n = x.shape[0]
    src, dst = edge_index[0], edge_index[1]
    xv = x[:, 0]
    ea = edge_attr[:, 0]
    w1 = W1.reshape(_H1, _C1)
    cs = (w1 * a1_src[0]).sum(-1)
    cd = (w1 * a1_dst[0]).sum(-1)
    we1 = We1.reshape(_H1, _C1)
    ce = (we1 * ae1[0]).sum(-1)
    maxabs_x = jnp.max(jnp.abs(xv))
    maxabs_ea = jnp.max(jnp.abs(ea))
    B1 = maxabs_x * jnp.abs(cs) + maxabs_ea * jnp.abs(ce)

    cnt = jnp.zeros((n,), jnp.float32).at[dst].add(1.0)
    ssum = jnp.zeros((n,), jnp.float32).at[dst].add(ea)
    loop_attr = ssum / jnp.clip(cnt, 1.0)

    xs = xv[src]
    xd = xv[dst]
    pre = xs[:, None] * cs + xd[:, None] * cd + ea[:, None] * ce
    M = _leaky(B1 + xd[:, None] * cd)
    e1 = jnp.exp(_leaky(pre) - M)
    den = jnp.zeros((n, _H1), jnp.float32).at[dst].add(e1)
    num = jnp.zeros((n, _H1), jnp.float32).at[dst].add(e1 * xs[:, None])

    pre_l = xv[:, None] * (cs + cd) + loop_attr[:, None] * ce
    Ml = _leaky(B1 + xv[:, None] * cd)
    el1 = jnp.exp(_leaky(pre_l) - Ml)
    s = (num + el1 * xv[:, None]) / (den + el1 + 1e-16)
    out1 = (s[:, :, None] * w1[None]).reshape(n, _H1 * _C1) + b1
    h2 = jax.nn.elu(out1)
    g = h2 @ W2
    a2s = (g * a2_src[0, 0]).sum(-1)
    a2d = (g * a2_dst[0, 0]).sum(-1)
    ce2 = (We2[0] * ae2[0, 0]).sum()
    maxabs_eaf = jnp.maximum(maxabs_ea, jnp.max(jnp.abs(loop_attr)))
    B2 = jnp.max(jnp.abs(a2s)) + maxabs_eaf * jnp.abs(ce2)

    pre2 = a2s[src] + a2d[dst] + ea * ce2
    M2 = _leaky(B2 + a2d[dst])
    e2 = jnp.exp(_leaky(pre2) - M2)
    den2 = jnp.zeros((n,), jnp.float32).at[dst].add(e2)
    num2 = jnp.zeros((n, 3), jnp.float32).at[dst].add(e2[:, None] * g[src])

    pre2l = a2s + a2d + loop_attr * ce2
    M2l = _leaky(B2 + a2d)
    e2l = jnp.exp(_leaky(pre2l) - M2l)

    out2 = pl.pallas_call(
        _final_combine_kernel,
        out_shape=jax.ShapeDtypeStruct((n, 3), jnp.float32),
    )(num2, den2[:, None], g, e2l[:, None], b2[None, :])
    return out2


# speedup vs baseline: 4.8726x; 4.8726x over previous
import jax
import jax.numpy as jnp
from jax.experimental import pallas as pl

_H1, _C1 = 8, 8


def _leaky(v):
    return jnp.where(v >= 0, v, 0.2 * v)


def _final_combine_kernel(num2_ref, den2_ref, g_ref, el_ref, b2_ref, out_ref):
    num2 = num2_ref[...]
    den2 = den2_ref[...]
    g = g_ref[...]
    el = el_ref[...]
    out_ref[...] = (num2 + el * g) / (den2 + el + 1e-16) + b2_ref[...]


def kernel(x, edge_index, edge_attr, W1, a1_src, a1_dst, We1, ae1, b1,
           W2, a2_src, a2_dst, We2, ae2, b2):
    n = x.shape[0]
    src, dst = edge_index[0], edge_index[1]
    xv = x[:, 0]
    ea = edge_attr[:, 0]
    w1 = W1.reshape(_H1, _C1)
    cs = (w1 * a1_src[0]).sum(-1)
    cd = (w1 * a1_dst[0]).sum(-1)
    we1 = We1.reshape(_H1, _C1)
    ce = (we1 * ae1[0]).sum(-1)
    maxabs_x = jnp.max(jnp.abs(xv))
    maxabs_ea = jnp.max(jnp.abs(ea))
    B1 = maxabs_x * jnp.abs(cs) + maxabs_ea * jnp.abs(ce)

    cnt = jnp.zeros((n,), jnp.float32).at[dst].add(1.0)
    ssum = jnp.zeros((n,), jnp.float32).at[dst].add(ea)
    loop_attr = ssum / jnp.clip(cnt, 1.0)

    xs = xv[src]
    xd = xv[dst]
    pre = xs[:, None] * cs + xd[:, None] * cd + ea[:, None] * ce
    M = _leaky(B1 + xd[:, None] * cd)
    e1 = jnp.exp(_leaky(pre) - M)
    den = jnp.zeros((n, _H1), jnp.float32).at[dst].add(e1)
    num = jnp.zeros((n, _H1), jnp.float32).at[dst].add(e1 * xs[:, None])

    pre_l = xv[:, None] * (cs + cd) + loop_attr[:, None] * ce
    Ml = _leaky(B1 + xv[:, None] * cd)
    el1 = jnp.exp(_leaky(pre_l) - Ml)
    s = (num + el1 * xv[:, None]) / (den + el1 + 1e-16)
    out1 = (s[:, :, None] * w1[None]).reshape(n, _H1 * _C1) + b1
    h2 = jax.nn.elu(out1)
    g = h2 @ W2
    a2s = (g * a2_src[0, 0]).sum(-1)
    a2d = (g * a2_dst[0, 0]).sum(-1)
    ce2 = (We2[0] * ae2[0, 0]).sum()
    maxabs_eaf = jnp.maximum(maxabs_ea, jnp.max(jnp.abs(loop_attr)))
    B2 = jnp.max(jnp.abs(a2s)) + maxabs_eaf * jnp.abs(ce2)

    pre2 = a2s[src] + a2d[dst] + ea * ce2
    M2 = _leaky(B2 + a2d[dst])
    e2 = jnp.exp(_leaky(pre2) - M2)
    den2 = jnp.zeros((n,), jnp.float32).at[dst].add(e2)
    num2 = jnp.zeros((n, 3), jnp.float32).at[dst].add(e2[:, None] * g[src])

    pre2l = a2s + a2d + loop_attr * ce2
    M2l = _leaky(B2 + a2d)
    e2l = jnp.exp(_leaky(pre2l) - M2l)

    rows = 2000
    grid = (n // rows,)
    row_spec = pl.BlockSpec((rows, 3), lambda i: (i, 0))
    col_spec = pl.BlockSpec((rows, 1), lambda i: (i, 0))
    out2 = pl.pallas_call(
        _final_combine_kernel,
        grid=grid,
        in_specs=[row_spec, col_spec, row_spec, col_spec,
                  pl.BlockSpec((1, 3), lambda i: (0, 0))],
        out_specs=row_spec,
        out_shape=jax.ShapeDtypeStruct((n, 3), jnp.float32),
    )(num2, den2[:, None], g, e2l[:, None], b2[None, :])
    return out2


# speedup vs baseline: 9.7467x; 2.0003x over previous
import functools

import jax
import jax.numpy as jnp
from jax import lax
from jax.experimental import pallas as pl
from jax.experimental.pallas import tpu as pltpu
from jax.experimental.pallas import tpu_sc as plsc

_H1, _C1 = 8, 8
_N = 50000
_E = 1600000

_NW = 32
_CH = 25
_K = 2048
_B = 128
_NB = _K // _B
_EPAD = _NW * _CH * _K
_NROW = 51200
_RPT = _NROW // 16


def _leaky(v):
    return jnp.where(v >= 0, v, 0.2 * v)


_sc_mesh = plsc.VectorSubcoreMesh(core_axis_name="c", subcore_axis_name="s")


@functools.partial(
    pl.kernel,
    out_type=(
        jax.ShapeDtypeStruct((2, _NROW, 16), jnp.float32),
        jax.ShapeDtypeStruct((2, _NROW), jnp.float32),
        jax.ShapeDtypeStruct((2, _NROW), jnp.float32),
    ),
    mesh=_sc_mesh,
    compiler_params=pltpu.CompilerParams(use_tc_tiling_on_sc=False),
    scratch_types=[
        pltpu.VMEM((_NB, _B), jnp.int32),
        pltpu.VMEM((_NB, _B), jnp.int32),
        pltpu.VMEM((_K,), jnp.float32),
        pltpu.VMEM((_K,), jnp.float32),
        pltpu.VMEM((_K,), jnp.float32),
        pltpu.VMEM((_K,), jnp.float32),
        pltpu.VMEM((_K, 16), jnp.float32),
        pltpu.VMEM((4, 16), jnp.float32),
        pltpu.VMEM_SHARED((_NROW,), jnp.float32),
        pltpu.VMEM_SHARED((_NROW, 16), jnp.float32),
        pltpu.VMEM_SHARED((_NROW,), jnp.float32),
        pltpu.VMEM_SHARED((_NROW,), jnp.float32),
    ],
)
def _k1(src2_hbm, dst2_hbm, ea_hbm, x_hbm, cbuf_hbm, ones_hbm, z16_hbm, z1_hbm,
        out_hbm, outC_hbm, outS_hbm,
        src2_v, dst2_v, ea_v, xs_v, xd_v, ones_v, S, cbuf_v, X, acc, accC, accS):
    cid = lax.axis_index("c")
    sid = lax.axis_index("s")
    wid = sid * 2 + cid
    pltpu.sync_copy(cbuf_hbm, cbuf_v)
    pltpu.sync_copy(ones_hbm, ones_v)
    r0 = sid * _RPT
    pltpu.sync_copy(x_hbm.at[pl.ds(r0, _RPT)], X.at[pl.ds(r0, _RPT)])
    pltpu.sync_copy(z16_hbm, acc.at[pl.ds(r0, _RPT)])
    pltpu.sync_copy(z1_hbm, accC.at[pl.ds(r0, _RPT)])
    pltpu.sync_copy(z1_hbm, accS.at[pl.ds(r0, _RPT)])
    plsc.subcore_barrier()

    iota = lax.iota(jnp.int32, 16)
    lane8 = iota < 8
    ones16 = jnp.ones((16,), jnp.float32)
    csv = cbuf_v[0, :]
    cdv = cbuf_v[1, :]
    cev = cbuf_v[2, :]
    b1v = cbuf_v[3, :]

    jb0 = wid * (_CH * _NB)

    def chunk(c, carry):
        jb = jb0 + c * _NB
        pltpu.sync_copy(src2_hbm.at[pl.ds(jb, _NB)], src2_v)
        pltpu.sync_copy(dst2_hbm.at[pl.ds(jb, _NB)], dst2_v)
        pltpu.sync_copy(ea_hbm.at[pl.ds(jb * _B, _K)], ea_v)

        def batch(j, bcarry):
            off0 = j * _B
            pltpu.sync_copy(X.at[src2_v.at[j]], xs_v.at[pl.ds(off0, _B)])
            pltpu.sync_copy(X.at[dst2_v.at[j]], xd_v.at[pl.ds(off0, _B)])

            def group(l, gcarry):
                off = off0 + l * 16
                xs16 = xs_v[pl.ds(off, 16)]
                xd16 = xd_v[pl.ds(off, 16)]
                ea16 = ea_v[pl.ds(off, 16)]
                for i in range(16):
                    bi = jnp.full((16,), i, jnp.int32)
                    xs_b = xs16.at[bi].get(mode="promise_in_bounds")
                    xd_b = xd16.at[bi].get(mode="promise_in_bounds")
                    ea_b = ea16.at[bi].get(mode="promise_in_bounds")
                    xdc = xd_b * cdv
                    t = xs_b * csv + xdc + ea_b * cev
                    al = jnp.where(t >= 0, t, 0.2 * t)
                    u = b1v + xdc
                    mh = jnp.where(u >= 0, u, 0.2 * u)
                    e = jnp.exp(al - mh)
                    row = e * jnp.where(lane8, ones16, xs_b)
                    S[off + i, :] = row
                return gcarry

            lax.fori_loop(0, _B // 16, group, 0)
            pltpu.sync_copy(S.at[pl.ds(off0, _B)], acc.at[dst2_v.at[j]], add=True)
            pltpu.sync_copy(ones_v.at[pl.ds(off0, _B)], accC.at[dst2_v.at[j]], add=True)
            pltpu.sync_copy(ea_v.at[pl.ds(off0, _B)], accS.at[dst2_v.at[j]], add=True)
            return bcarry

        lax.fori_loop(0, _NB, batch, 0)
        return carry

    lax.fori_loop(0, _CH, chunk, 0)
    plsc.subcore_barrier()
    pltpu.sync_copy(acc.at[pl.ds(r0, _RPT)], out_hbm.at[cid, pl.ds(r0, _RPT)])
    pltpu.sync_copy(accC.at[pl.ds(r0, _RPT)], outC_hbm.at[cid, pl.ds(r0, _RPT)])
    pltpu.sync_copy(accS.at[pl.ds(r0, _RPT)], outS_hbm.at[cid, pl.ds(r0, _RPT)])


def _final_combine_kernel(num2_ref, den2_ref, g_ref, el_ref, b2_ref, out_ref):
    out_ref[...] = ((num2_ref[...] + el_ref[...] * g_ref[...])
                    / (den2_ref[...] + el_ref[...] + 1e-16) + b2_ref[...])


def kernel(x, edge_index, edge_attr, W1, a1_src, a1_dst, We1, ae1, b1,
           W2, a2_src, a2_dst, We2, ae2, b2):
    n = x.shape[0]
    src, dst = edge_index[0], edge_index[1]
    src = src.astype(jnp.int32)
    dst = dst.astype(jnp.int32)
    xv = x[:, 0]
    ea = edge_attr[:, 0]
    w1 = W1.reshape(_H1, _C1)
    cs = (w1 * a1_src[0]).sum(-1)
    cd = (w1 * a1_dst[0]).sum(-1)
    we1 = We1.reshape(_H1, _C1)
    ce = (we1 * ae1[0]).sum(-1)
    maxabs_x = jnp.max(jnp.abs(xv))
    maxabs_ea = jnp.max(jnp.abs(ea))
    B1 = maxabs_x * jnp.abs(cs) + maxabs_ea * jnp.abs(ce)

    npad = _EPAD - _E
    src_r = jnp.concatenate([src, jnp.zeros((npad,), jnp.int32)]).reshape(_EPAD // _B, _B)
    dst_r = jnp.concatenate([dst, jnp.full((npad,), _N, jnp.int32)]).reshape(_EPAD // _B, _B)
    ea_p = jnp.concatenate([ea, jnp.zeros((npad,), jnp.float32)])
    x_p = jnp.concatenate([xv, jnp.zeros((_NROW - _N,), jnp.float32)])
    cbuf = jnp.stack([
        jnp.tile(cs, 2), jnp.tile(cd, 2), jnp.tile(ce, 2), jnp.tile(B1, 2),
    ])
    onesk = jnp.ones((_K,), jnp.float32)
    z16 = jnp.zeros((_RPT, 16), jnp.float32)
    z1 = jnp.zeros((_RPT,), jnp.float32)

    part, partC, partS = _k1(src_r, dst_r, ea_p, x_p, cbuf, onesk, z16, z1)
    den = part[0, :n, 0:8] + part[1, :n, 0:8]
    num = part[0, :n, 8:16] + part[1, :n, 8:16]
    cnt = partC[0, :n] + partC[1, :n]
    ssum = partS[0, :n] + partS[1, :n]
    loop_attr = ssum / jnp.clip(cnt, 1.0)

    pre_l = xv[:, None] * (cs + cd) + loop_attr[:, None] * ce
    Ml = _leaky(B1 + xv[:, None] * cd)
    el1 = jnp.exp(_leaky(pre_l) - Ml)
    s = (num + el1 * xv[:, None]) / (den + el1 + 1e-16)
    out1 = (s[:, :, None] * w1[None]).reshape(n, _H1 * _C1) + b1
    h2 = jax.nn.elu(out1)
    g = h2 @ W2
    a2s = (g * a2_src[0, 0]).sum(-1)
    a2d = (g * a2_dst[0, 0]).sum(-1)
    ce2 = (We2[0] * ae2[0, 0]).sum()
    maxabs_eaf = jnp.maximum(maxabs_ea, jnp.max(jnp.abs(loop_attr)))
    B2 = jnp.max(jnp.abs(a2s)) + maxabs_eaf * jnp.abs(ce2)

    pre2 = a2s[src] + a2d[dst] + ea * ce2
    M2 = _leaky(B2 + a2d[dst])
    e2 = jnp.exp(_leaky(pre2) - M2)
    den2 = jnp.zeros((n,), jnp.float32).at[dst].add(e2)
    num2 = jnp.zeros((n, 3), jnp.float32).at[dst].add(e2[:, None] * g[src])

    pre2l = a2s + a2d + loop_attr * ce2
    M2l = _leaky(B2 + a2d)
    e2l = jnp.exp(_leaky(pre2l) - M2l)

    rows = 2000
    grid = (n // rows,)
    row_spec = pl.BlockSpec((rows, 3), lambda i: (i, 0))
    col_spec = pl.BlockSpec((rows, 1), lambda i: (i, 0))
    out2 = pl.pallas_call(
        _final_combine_kernel,
        grid=grid,
        in_specs=[row_spec, col_spec, row_spec, col_spec,
                  pl.BlockSpec((1, 3), lambda i: (0, 0))],
        out_specs=row_spec,
        out_shape=jax.ShapeDtypeStruct((n, 3), jnp.float32),
    )(num2, den2[:, None], g, e2l[:, None], b2[None, :])
    return out2


# speedup vs baseline: 194.2390x; 19.9287x over previous
import functools

import jax
import jax.numpy as jnp
from jax import lax
from jax.experimental import pallas as pl
from jax.experimental.pallas import tpu as pltpu
from jax.experimental.pallas import tpu_sc as plsc

_H1, _C1 = 8, 8
_N = 50000
_E = 1600000

_NW = 32
_CH = 25
_K = 2048
_B = 128
_NB = _K // _B
_EPAD = _NW * _CH * _K
_NROW = 51200
_RPT = _NROW // 16


def _leaky(v):
    return jnp.where(v >= 0, v, 0.2 * v)


_sc_mesh = plsc.VectorSubcoreMesh(core_axis_name="c", subcore_axis_name="s")


@functools.partial(
    pl.kernel,
    out_type=(
        jax.ShapeDtypeStruct((2, _NROW, 16), jnp.float32),
        jax.ShapeDtypeStruct((2, _NROW), jnp.float32),
        jax.ShapeDtypeStruct((2, _NROW), jnp.float32),
    ),
    mesh=_sc_mesh,
    compiler_params=pltpu.CompilerParams(use_tc_tiling_on_sc=False),
    scratch_types=[
        pltpu.VMEM((_NB, _B), jnp.int32),
        pltpu.VMEM((_NB, _B), jnp.int32),
        pltpu.VMEM((_K,), jnp.float32),
        pltpu.VMEM((_K,), jnp.float32),
        pltpu.VMEM((_K,), jnp.float32),
        pltpu.VMEM((_K,), jnp.float32),
        pltpu.VMEM((_K, 16), jnp.float32),
        pltpu.VMEM((4, 16), jnp.float32),
        pltpu.VMEM_SHARED((_NROW,), jnp.float32),
        pltpu.VMEM_SHARED((_NROW, 16), jnp.float32),
        pltpu.VMEM_SHARED((_NROW,), jnp.float32),
        pltpu.VMEM_SHARED((_NROW,), jnp.float32),
    ],
)
def _k1(src2_hbm, dst2_hbm, ea_hbm, x_hbm, cbuf_hbm, ones_hbm, z16_hbm, z1_hbm,
        out_hbm, outC_hbm, outS_hbm,
        src2_v, dst2_v, ea_v, xs_v, xd_v, ones_v, S, cbuf_v, X, acc, accC, accS):
    cid = lax.axis_index("c")
    sid = lax.axis_index("s")
    wid = sid * 2 + cid
    pltpu.sync_copy(cbuf_hbm, cbuf_v)
    pltpu.sync_copy(ones_hbm, ones_v)
    r0 = sid * _RPT
    pltpu.sync_copy(x_hbm.at[pl.ds(r0, _RPT)], X.at[pl.ds(r0, _RPT)])
    pltpu.sync_copy(z16_hbm, acc.at[pl.ds(r0, _RPT)])
    pltpu.sync_copy(z1_hbm, accC.at[pl.ds(r0, _RPT)])
    pltpu.sync_copy(z1_hbm, accS.at[pl.ds(r0, _RPT)])
    plsc.subcore_barrier()

    iota = lax.iota(jnp.int32, 16)
    lane8 = iota < 8
    ones16 = jnp.ones((16,), jnp.float32)
    csv = cbuf_v[0, :]
    cdv = cbuf_v[1, :]
    cev = cbuf_v[2, :]
    b1v = cbuf_v[3, :]

    jb0 = wid * (_CH * _NB)

    def chunk(c, carry):
        jb = jb0 + c * _NB
        pltpu.sync_copy(src2_hbm.at[pl.ds(jb, _NB)], src2_v)
        pltpu.sync_copy(dst2_hbm.at[pl.ds(jb, _NB)], dst2_v)
        pltpu.sync_copy(ea_hbm.at[pl.ds(jb * _B, _K)], ea_v)

        def batch(j, bcarry):
            off0 = j * _B
            pltpu.sync_copy(X.at[src2_v.at[j]], xs_v.at[pl.ds(off0, _B)])
            pltpu.sync_copy(X.at[dst2_v.at[j]], xd_v.at[pl.ds(off0, _B)])

            def group(l, gcarry):
                off = off0 + l * 16
                xs16 = xs_v[pl.ds(off, 16)]
                xd16 = xd_v[pl.ds(off, 16)]
                ea16 = ea_v[pl.ds(off, 16)]
                for i in range(16):
                    bi = jnp.full((16,), i, jnp.int32)
                    xs_b = xs16.at[bi].get(mode="promise_in_bounds")
                    xd_b = xd16.at[bi].get(mode="promise_in_bounds")
                    ea_b = ea16.at[bi].get(mode="promise_in_bounds")
                    xdc = xd_b * cdv
                    t = xs_b * csv + xdc + ea_b * cev
                    al = jnp.where(t >= 0, t, 0.2 * t)
                    u = b1v + xdc
                    mh = jnp.where(u >= 0, u, 0.2 * u)
                    e = jnp.exp(al - mh)
                    row = e * jnp.where(lane8, ones16, xs_b)
                    S[off + i, :] = row
                return gcarry

            lax.fori_loop(0, _B // 16, group, 0)
            pltpu.sync_copy(S.at[pl.ds(off0, _B)], acc.at[dst2_v.at[j]], add=True)
            pltpu.sync_copy(ones_v.at[pl.ds(off0, _B)], accC.at[dst2_v.at[j]], add=True)
            pltpu.sync_copy(ea_v.at[pl.ds(off0, _B)], accS.at[dst2_v.at[j]], add=True)
            return bcarry

        lax.fori_loop(0, _NB, batch, 0)
        return carry

    lax.fori_loop(0, _CH, chunk, 0)
    plsc.subcore_barrier()
    pltpu.sync_copy(acc.at[pl.ds(r0, _RPT)], out_hbm.at[cid, pl.ds(r0, _RPT)])
    pltpu.sync_copy(accC.at[pl.ds(r0, _RPT)], outC_hbm.at[cid, pl.ds(r0, _RPT)])
    pltpu.sync_copy(accS.at[pl.ds(r0, _RPT)], outS_hbm.at[cid, pl.ds(r0, _RPT)])


@functools.partial(
    pl.kernel,
    out_type=jax.ShapeDtypeStruct((2, 4, _NROW), jnp.float32),
    mesh=_sc_mesh,
    compiler_params=pltpu.CompilerParams(use_tc_tiling_on_sc=False),
    scratch_types=[
        pltpu.VMEM((_NB, _B), jnp.int32),
        pltpu.VMEM((_NB, _B), jnp.int32),
        pltpu.VMEM((_K,), jnp.float32),
        pltpu.VMEM((_K,), jnp.float32),
        pltpu.VMEM((_K,), jnp.float32),
        pltpu.VMEM((_K,), jnp.float32),
        pltpu.VMEM((_K,), jnp.float32),
        pltpu.VMEM((_K,), jnp.float32),
        pltpu.VMEM((_K,), jnp.float32),
        pltpu.VMEM((_K,), jnp.float32),
        pltpu.VMEM((_K,), jnp.float32),
        pltpu.VMEM((_K,), jnp.float32),
        pltpu.VMEM((2, 16), jnp.float32),
        pltpu.VMEM_SHARED((_NROW,), jnp.float32),
        pltpu.VMEM_SHARED((_NROW,), jnp.float32),
        pltpu.VMEM_SHARED((_NROW,), jnp.float32),
        pltpu.VMEM_SHARED((_NROW,), jnp.float32),
        pltpu.VMEM_SHARED((_NROW,), jnp.float32),
        pltpu.VMEM_SHARED((_NROW,), jnp.float32),
        pltpu.VMEM_SHARED((_NROW,), jnp.float32),
        pltpu.VMEM_SHARED((_NROW,), jnp.float32),
        pltpu.VMEM_SHARED((_NROW,), jnp.float32),
    ],
)
def _k3(src2_hbm, dst2_hbm, ea_hbm, a2s_hbm, g0_hbm, g1_hbm, g2_hbm, a2d_hbm,
        cbuf_hbm, z1_hbm, out_hbm,
        src2_v, dst2_v, ea_v, as_v, g0_v, g1_v, g2_v, ad_v,
        e_v, m0_v, m1_v, m2_v, cbuf_v,
        AS, G0, G1, G2, AD, den2, num0, num1, num2):
    cid = lax.axis_index("c")
    sid = lax.axis_index("s")
    wid = sid * 2 + cid
    pltpu.sync_copy(cbuf_hbm, cbuf_v)
    r0 = sid * _RPT
    sl = pl.ds(r0, _RPT)
    pltpu.sync_copy(a2s_hbm.at[sl], AS.at[sl])
    pltpu.sync_copy(g0_hbm.at[sl], G0.at[sl])
    pltpu.sync_copy(g1_hbm.at[sl], G1.at[sl])
    pltpu.sync_copy(g2_hbm.at[sl], G2.at[sl])
    pltpu.sync_copy(a2d_hbm.at[sl], AD.at[sl])
    pltpu.sync_copy(z1_hbm, den2.at[sl])
    pltpu.sync_copy(z1_hbm, num0.at[sl])
    pltpu.sync_copy(z1_hbm, num1.at[sl])
    pltpu.sync_copy(z1_hbm, num2.at[sl])
    plsc.subcore_barrier()

    ce2v = cbuf_v[0, :]
    b2v = cbuf_v[1, :]
    jb0 = wid * (_CH * _NB)

    def chunk(c, carry):
        jb = jb0 + c * _NB
        pltpu.sync_copy(src2_hbm.at[pl.ds(jb, _NB)], src2_v)
        pltpu.sync_copy(dst2_hbm.at[pl.ds(jb, _NB)], dst2_v)
        pltpu.sync_copy(ea_hbm.at[pl.ds(jb * _B, _K)], ea_v)

        def batch(j, bcarry):
            off0 = j * _B
            sb = pl.ds(off0, _B)
            pltpu.sync_copy(AS.at[src2_v.at[j]], as_v.at[sb])
            pltpu.sync_copy(G0.at[src2_v.at[j]], g0_v.at[sb])
            pltpu.sync_copy(G1.at[src2_v.at[j]], g1_v.at[sb])
            pltpu.sync_copy(G2.at[src2_v.at[j]], g2_v.at[sb])
            pltpu.sync_copy(AD.at[dst2_v.at[j]], ad_v.at[sb])

            def group(l, gcarry):
                off = off0 + l * 16
                so = pl.ds(off, 16)
                as16 = as_v[so]
                ad16 = ad_v[so]
                ea16 = ea_v[so]
                t = as16 + ad16 + ea16 * ce2v
                al = jnp.where(t >= 0, t, 0.2 * t)
                u = b2v + ad16
                mh = jnp.where(u >= 0, u, 0.2 * u)
                e2 = jnp.exp(al - mh)
                e_v[so] = e2
                m0_v[so] = e2 * g0_v[so]
                m1_v[so] = e2 * g1_v[so]
                m2_v[so] = e2 * g2_v[so]
                return gcarry

            lax.fori_loop(0, _B // 16, group, 0)
            pltpu.sync_copy(e_v.at[sb], den2.at[dst2_v.at[j]], add=True)
            pltpu.sync_copy(m0_v.at[sb], num0.at[dst2_v.at[j]], add=True)
            pltpu.sync_copy(m1_v.at[sb], num1.at[dst2_v.at[j]], add=True)
            pltpu.sync_copy(m2_v.at[sb], num2.at[dst2_v.at[j]], add=True)
            return bcarry

        lax.fori_loop(0, _NB, batch, 0)
        return carry

    lax.fori_loop(0, _CH, chunk, 0)
    plsc.subcore_barrier()
    pltpu.sync_copy(den2.at[sl], out_hbm.at[cid, 0, sl])
    pltpu.sync_copy(num0.at[sl], out_hbm.at[cid, 1, sl])
    pltpu.sync_copy(num1.at[sl], out_hbm.at[cid, 2, sl])
    pltpu.sync_copy(num2.at[sl], out_hbm.at[cid, 3, sl])


def _final_combine_kernel(num2_ref, den2_ref, g_ref, el_ref, b2_ref, out_ref):
    out_ref[...] = ((num2_ref[...] + el_ref[...] * g_ref[...])
                    / (den2_ref[...] + el_ref[...] + 1e-16) + b2_ref[...])


def kernel(x, edge_index, edge_attr, W1, a1_src, a1_dst, We1, ae1, b1,
           W2, a2_src, a2_dst, We2, ae2, b2):
    n = x.shape[0]
    src, dst = edge_index[0], edge_index[1]
    src = src.astype(jnp.int32)
    dst = dst.astype(jnp.int32)
    xv = x[:, 0]
    ea = edge_attr[:, 0]
    w1 = W1.reshape(_H1, _C1)
    cs = (w1 * a1_src[0]).sum(-1)
    cd = (w1 * a1_dst[0]).sum(-1)
    we1 = We1.reshape(_H1, _C1)
    ce = (we1 * ae1[0]).sum(-1)
    maxabs_x = jnp.max(jnp.abs(xv))
    maxabs_ea = jnp.max(jnp.abs(ea))
    B1 = maxabs_x * jnp.abs(cs) + maxabs_ea * jnp.abs(ce)

    npad = _EPAD - _E
    src_r = jnp.concatenate([src, jnp.zeros((npad,), jnp.int32)]).reshape(_EPAD // _B, _B)
    dst_r = jnp.concatenate([dst, jnp.full((npad,), _N, jnp.int32)]).reshape(_EPAD // _B, _B)
    ea_p = jnp.concatenate([ea, jnp.zeros((npad,), jnp.float32)])
    x_p = jnp.concatenate([xv, jnp.zeros((_NROW - _N,), jnp.float32)])
    cbuf = jnp.stack([
        jnp.tile(cs, 2), jnp.tile(cd, 2), jnp.tile(ce, 2), jnp.tile(B1, 2),
    ])
    onesk = jnp.ones((_K,), jnp.float32)
    z16 = jnp.zeros((_RPT, 16), jnp.float32)
    z1 = jnp.zeros((_RPT,), jnp.float32)

    part, partC, partS = _k1(src_r, dst_r, ea_p, x_p, cbuf, onesk, z16, z1)
    den = part[0, :n, 0:8] + part[1, :n, 0:8]
    num = part[0, :n, 8:16] + part[1, :n, 8:16]
    cnt = partC[0, :n] + partC[1, :n]
    ssum = partS[0, :n] + partS[1, :n]
    loop_attr = ssum / jnp.clip(cnt, 1.0)

    pre_l = xv[:, None] * (cs + cd) + loop_attr[:, None] * ce
    Ml = _leaky(B1 + xv[:, None] * cd)
    el1 = jnp.exp(_leaky(pre_l) - Ml)
    s = (num + el1 * xv[:, None]) / (den + el1 + 1e-16)
    out1 = (s[:, :, None] * w1[None]).reshape(n, _H1 * _C1) + b1
    h2 = jax.nn.elu(out1)
    g = h2 @ W2
    a2s = (g * a2_src[0, 0]).sum(-1)
    a2d = (g * a2_dst[0, 0]).sum(-1)
    ce2 = (We2[0] * ae2[0, 0]).sum()
    maxabs_eaf = jnp.maximum(maxabs_ea, jnp.max(jnp.abs(loop_attr)))
    B2 = jnp.max(jnp.abs(a2s)) + maxabs_eaf * jnp.abs(ce2)

    zpad = jnp.zeros((_NROW - _N,), jnp.float32)
    cbuf2 = jnp.stack([jnp.full((16,), ce2), jnp.full((16,), B2)])
    p2 = _k3(src_r, dst_r, ea_p,
             jnp.concatenate([a2s, zpad]),
             jnp.concatenate([g[:, 0], zpad]),
             jnp.concatenate([g[:, 1], zpad]),
             jnp.concatenate([g[:, 2], zpad]),
             jnp.concatenate([a2d, zpad]),
             cbuf2, z1)
    den2 = p2[0, 0, :n] + p2[1, 0, :n]
    num2 = (p2[0, 1:4, :n] + p2[1, 1:4, :n]).T

    pre2l = a2s + a2d + loop_attr * ce2
    M2l = _leaky(B2 + a2d)
    e2l = jnp.exp(_leaky(pre2l) - M2l)

    rows = 2000
    grid = (n // rows,)
    row_spec = pl.BlockSpec((rows, 3), lambda i: (i, 0))
    col_spec = pl.BlockSpec((rows, 1), lambda i: (i, 0))
    out2 = pl.pallas_call(
        _final_combine_kernel,
        grid=grid,
        in_specs=[row_spec, col_spec, row_spec, col_spec,
                  pl.BlockSpec((1, 3), lambda i: (0, 0))],
        out_specs=row_spec,
        out_shape=jax.ShapeDtypeStruct((n, 3), jnp.float32),
    )(num2, den2[:, None], g, e2l[:, None], b2[None, :])
    return out2


# speedup vs baseline: 336.0711x; 1.7302x over previous
import functools

import jax
import jax.numpy as jnp
from jax import lax
from jax.experimental import pallas as pl
from jax.experimental.pallas import tpu as pltpu
from jax.experimental.pallas import tpu_sc as plsc

_H1, _C1 = 8, 8
_N = 50000
_E = 1600000

_NW = 32
_CH = 25
_K = 2000
_B = 80
_NB = _K // _B
_NROW = 51200
_RPT = _NROW // 16


def _leaky(v):
    return jnp.where(v >= 0, v, 0.2 * v)


_sc_mesh = plsc.VectorSubcoreMesh(core_axis_name="c", subcore_axis_name="s")


@functools.partial(
    pl.kernel,
    out_type=(
        jax.ShapeDtypeStruct((2, _NROW, 16), jnp.float32),
        jax.ShapeDtypeStruct((2, _NROW), jnp.float32),
        jax.ShapeDtypeStruct((2, _NROW), jnp.float32),
    ),
    mesh=_sc_mesh,
    compiler_params=pltpu.CompilerParams(use_tc_tiling_on_sc=False),
    scratch_types=[
        pltpu.VMEM((_NB, _B), jnp.int32),
        pltpu.VMEM((_NB, _B), jnp.int32),
        pltpu.VMEM((_K,), jnp.float32),
        pltpu.VMEM((_K,), jnp.float32),
        pltpu.VMEM((_K,), jnp.float32),
        pltpu.VMEM((_K,), jnp.float32),
        pltpu.VMEM((_K, 16), jnp.float32),
        pltpu.VMEM((4, 16), jnp.float32),
        pltpu.VMEM_SHARED((_NROW,), jnp.float32),
        pltpu.VMEM_SHARED((_NROW, 16), jnp.float32),
        pltpu.VMEM_SHARED((_NROW,), jnp.float32),
        pltpu.VMEM_SHARED((_NROW,), jnp.float32),
        pltpu.SemaphoreType.DMA,
        pltpu.SemaphoreType.DMA,
    ],
)
def _k1(src2_hbm, dst2_hbm, ea_hbm, x_hbm, cbuf_hbm, ones_hbm, z16_hbm, z1_hbm,
        out_hbm, outC_hbm, outS_hbm,
        src2_v, dst2_v, ea_v, xs_v, xd_v, ones_v, S, cbuf_v, X, acc, accC, accS,
        sem_g, sem_s):
    cid = lax.axis_index("c")
    sid = lax.axis_index("s")
    wid = sid * 2 + cid
    pltpu.sync_copy(cbuf_hbm, cbuf_v)
    pltpu.sync_copy(ones_hbm, ones_v)
    r0 = sid * _RPT
    pltpu.sync_copy(x_hbm.at[pl.ds(r0, _RPT)], X.at[pl.ds(r0, _RPT)])
    pltpu.sync_copy(z16_hbm, acc.at[pl.ds(r0, _RPT)])
    pltpu.sync_copy(z1_hbm, accC.at[pl.ds(r0, _RPT)])
    pltpu.sync_copy(z1_hbm, accS.at[pl.ds(r0, _RPT)])
    plsc.subcore_barrier()

    iota = lax.iota(jnp.int32, 16)
    lane8 = iota < 8
    ones16 = jnp.ones((16,), jnp.float32)
    csv = cbuf_v[0, :]
    cdv = cbuf_v[1, :]
    cev = cbuf_v[2, :]
    b1v = cbuf_v[3, :]

    jb0 = wid * (_CH * _NB)

    def chunk(c, carry):
        jb = jb0 + c * _NB
        pltpu.sync_copy(src2_hbm.at[pl.ds(jb, _NB)], src2_v)
        pltpu.sync_copy(dst2_hbm.at[pl.ds(jb, _NB)], dst2_v)
        pltpu.sync_copy(ea_hbm.at[pl.ds(jb * _B, _K)], ea_v)

        def gathers(j):
            off0 = j * _B
            return (
                pltpu.async_copy(X.at[src2_v.at[j]], xs_v.at[pl.ds(off0, _B)], sem_g),
                pltpu.async_copy(X.at[dst2_v.at[j]], xd_v.at[pl.ds(off0, _B)], sem_g),
            )

        def scatters(j):
            off0 = j * _B
            return (
                pltpu.async_copy(S.at[pl.ds(off0, _B)], acc.at[dst2_v.at[j]], sem_s, add=True),
                pltpu.async_copy(ones_v.at[pl.ds(off0, _B)], accC.at[dst2_v.at[j]], sem_s, add=True),
                pltpu.async_copy(ea_v.at[pl.ds(off0, _B)], accS.at[dst2_v.at[j]], sem_s, add=True),
            )

        g_pend = {0: gathers(0)}
        s_pend = {}
        for j in range(_NB):
            if j + 1 < _NB:
                g_pend[j + 1] = gathers(j + 1)
            for d in g_pend.pop(j):
                d.wait()
            off0 = j * _B

            def group(l, gcarry, off0=off0):
                off = off0 + l * 16
                xs16 = xs_v[pl.ds(off, 16)]
                xd16 = xd_v[pl.ds(off, 16)]
                ea16 = ea_v[pl.ds(off, 16)]
                for i in range(16):
                    bi = jnp.full((16,), i, jnp.int32)
                    xs_b = xs16.at[bi].get(mode="promise_in_bounds")
                    xd_b = xd16.at[bi].get(mode="promise_in_bounds")
                    ea_b = ea16.at[bi].get(mode="promise_in_bounds")
                    xdc = xd_b * cdv
                    t = xs_b * csv + xdc + ea_b * cev
                    al = jnp.where(t >= 0, t, 0.2 * t)
                    u = b1v + xdc
                    mh = jnp.where(u >= 0, u, 0.2 * u)
                    e = jnp.exp(al - mh)
                    row = e * jnp.where(lane8, ones16, xs_b)
                    S[off + i, :] = row
                return gcarry

            lax.fori_loop(0, _B // 16, group, 0)
            s_pend[j] = scatters(j)
            if j >= 4:
                for d in s_pend.pop(j - 4):
                    d.wait()
        for js in sorted(s_pend):
            for d in s_pend[js]:
                d.wait()
        return carry

    lax.fori_loop(0, _CH, chunk, 0)
    plsc.subcore_barrier()
    pltpu.sync_copy(acc.at[pl.ds(r0, _RPT)], out_hbm.at[cid, pl.ds(r0, _RPT)])
    pltpu.sync_copy(accC.at[pl.ds(r0, _RPT)], outC_hbm.at[cid, pl.ds(r0, _RPT)])
    pltpu.sync_copy(accS.at[pl.ds(r0, _RPT)], outS_hbm.at[cid, pl.ds(r0, _RPT)])


@functools.partial(
    pl.kernel,
    out_type=jax.ShapeDtypeStruct((2, 4, _NROW), jnp.float32),
    mesh=_sc_mesh,
    compiler_params=pltpu.CompilerParams(use_tc_tiling_on_sc=False),
    scratch_types=[
        pltpu.VMEM((_NB, _B), jnp.int32),
        pltpu.VMEM((_NB, _B), jnp.int32),
        pltpu.VMEM((_K,), jnp.float32),
        pltpu.VMEM((_K,), jnp.float32),
        pltpu.VMEM((_K,), jnp.float32),
        pltpu.VMEM((_K,), jnp.float32),
        pltpu.VMEM((_K,), jnp.float32),
        pltpu.VMEM((_K,), jnp.float32),
        pltpu.VMEM((_K,), jnp.float32),
        pltpu.VMEM((_K,), jnp.float32),
        pltpu.VMEM((_K,), jnp.float32),
        pltpu.VMEM((_K,), jnp.float32),
        pltpu.VMEM((2, 16), jnp.float32),
        pltpu.VMEM_SHARED((_NROW,), jnp.float32),
        pltpu.VMEM_SHARED((_NROW,), jnp.float32),
        pltpu.VMEM_SHARED((_NROW,), jnp.float32),
        pltpu.VMEM_SHARED((_NROW,), jnp.float32),
        pltpu.VMEM_SHARED((_NROW,), jnp.float32),
        pltpu.VMEM_SHARED((_NROW,), jnp.float32),
        pltpu.VMEM_SHARED((_NROW,), jnp.float32),
        pltpu.VMEM_SHARED((_NROW,), jnp.float32),
        pltpu.VMEM_SHARED((_NROW,), jnp.float32),
        pltpu.SemaphoreType.DMA,
        pltpu.SemaphoreType.DMA,
    ],
)
def _k3(src2_hbm, dst2_hbm, ea_hbm, a2s_hbm, g0_hbm, g1_hbm, g2_hbm, a2d_hbm,
        cbuf_hbm, z1_hbm, out_hbm,
        src2_v, dst2_v, ea_v, as_v, g0_v, g1_v, g2_v, ad_v,
        e_v, m0_v, m1_v, m2_v, cbuf_v,
        AS, G0, G1, G2, AD, den2, num0, num1, num2,
        sem_g, sem_s):
    cid = lax.axis_index("c")
    sid = lax.axis_index("s")
    wid = sid * 2 + cid
    pltpu.sync_copy(cbuf_hbm, cbuf_v)
    r0 = sid * _RPT
    sl = pl.ds(r0, _RPT)
    pltpu.sync_copy(a2s_hbm.at[sl], AS.at[sl])
    pltpu.sync_copy(g0_hbm.at[sl], G0.at[sl])
    pltpu.sync_copy(g1_hbm.at[sl], G1.at[sl])
    pltpu.sync_copy(g2_hbm.at[sl], G2.at[sl])
    pltpu.sync_copy(a2d_hbm.at[sl], AD.at[sl])
    pltpu.sync_copy(z1_hbm, den2.at[sl])
    pltpu.sync_copy(z1_hbm, num0.at[sl])
    pltpu.sync_copy(z1_hbm, num1.at[sl])
    pltpu.sync_copy(z1_hbm, num2.at[sl])
    plsc.subcore_barrier()

    ce2v = cbuf_v[0, :]
    b2v = cbuf_v[1, :]
    jb0 = wid * (_CH * _NB)

    def chunk(c, carry):
        jb = jb0 + c * _NB
        pltpu.sync_copy(src2_hbm.at[pl.ds(jb, _NB)], src2_v)
        pltpu.sync_copy(dst2_hbm.at[pl.ds(jb, _NB)], dst2_v)
        pltpu.sync_copy(ea_hbm.at[pl.ds(jb * _B, _K)], ea_v)

        def gathers(j):
            sb = pl.ds(j * _B, _B)
            return (
                pltpu.async_copy(AS.at[src2_v.at[j]], as_v.at[sb], sem_g),
                pltpu.async_copy(G0.at[src2_v.at[j]], g0_v.at[sb], sem_g),
                pltpu.async_copy(G1.at[src2_v.at[j]], g1_v.at[sb], sem_g),
                pltpu.async_copy(G2.at[src2_v.at[j]], g2_v.at[sb], sem_g),
                pltpu.async_copy(AD.at[dst2_v.at[j]], ad_v.at[sb], sem_g),
            )

        def scatters(j):
            sb = pl.ds(j * _B, _B)
            return (
                pltpu.async_copy(e_v.at[sb], den2.at[dst2_v.at[j]], sem_s, add=True),
                pltpu.async_copy(m0_v.at[sb], num0.at[dst2_v.at[j]], sem_s, add=True),
                pltpu.async_copy(m1_v.at[sb], num1.at[dst2_v.at[j]], sem_s, add=True),
                pltpu.async_copy(m2_v.at[sb], num2.at[dst2_v.at[j]], sem_s, add=True),
            )

        g_pend = {0: gathers(0)}
        s_pend = {}
        for j in range(_NB):
            if j + 1 < _NB:
                g_pend[j + 1] = gathers(j + 1)
            for d in g_pend.pop(j):
                d.wait()
            off0 = j * _B

            def group(l, gcarry, off0=off0):
                off = off0 + l * 16
                so = pl.ds(off, 16)
                as16 = as_v[so]
                ad16 = ad_v[so]
                ea16 = ea_v[so]
                t = as16 + ad16 + ea16 * ce2v
                al = jnp.where(t >= 0, t, 0.2 * t)
                u = b2v + ad16
                mh = jnp.where(u >= 0, u, 0.2 * u)
                e2 = jnp.exp(al - mh)
                e_v[so] = e2
                m0_v[so] = e2 * g0_v[so]
                m1_v[so] = e2 * g1_v[so]
                m2_v[so] = e2 * g2_v[so]
                return gcarry

            lax.fori_loop(0, _B // 16, group, 0)
            s_pend[j] = scatters(j)
            if j >= 4:
                for d in s_pend.pop(j - 4):
                    d.wait()
        for js in sorted(s_pend):
            for d in s_pend[js]:
                d.wait()
        return carry

    lax.fori_loop(0, _CH, chunk, 0)
    plsc.subcore_barrier()
    pltpu.sync_copy(den2.at[sl], out_hbm.at[cid, 0, sl])
    pltpu.sync_copy(num0.at[sl], out_hbm.at[cid, 1, sl])
    pltpu.sync_copy(num1.at[sl], out_hbm.at[cid, 2, sl])
    pltpu.sync_copy(num2.at[sl], out_hbm.at[cid, 3, sl])


def _final_combine_kernel(num2_ref, den2_ref, g_ref, el_ref, b2_ref, out_ref):
    out_ref[...] = ((num2_ref[...] + el_ref[...] * g_ref[...])
                    / (den2_ref[...] + el_ref[...] + 1e-16) + b2_ref[...])


def kernel(x, edge_index, edge_attr, W1, a1_src, a1_dst, We1, ae1, b1,
           W2, a2_src, a2_dst, We2, ae2, b2):
    n = x.shape[0]
    src, dst = edge_index[0], edge_index[1]
    src = src.astype(jnp.int32)
    dst = dst.astype(jnp.int32)
    xv = x[:, 0]
    ea = edge_attr[:, 0]
    w1 = W1.reshape(_H1, _C1)
    cs = (w1 * a1_src[0]).sum(-1)
    cd = (w1 * a1_dst[0]).sum(-1)
    we1 = We1.reshape(_H1, _C1)
    ce = (we1 * ae1[0]).sum(-1)
    maxabs_x = jnp.max(jnp.abs(xv))
    maxabs_ea = jnp.max(jnp.abs(ea))
    B1 = maxabs_x * jnp.abs(cs) + maxabs_ea * jnp.abs(ce)

    src_r = src.reshape(_E // _B, _B)
    dst_r = dst.reshape(_E // _B, _B)
    ea_p = ea
    x_p = jnp.concatenate([xv, jnp.zeros((_NROW - _N,), jnp.float32)])
    cbuf = jnp.stack([
        jnp.tile(cs, 2), jnp.tile(cd, 2), jnp.tile(ce, 2), jnp.tile(B1, 2),
    ])
    onesk = jnp.ones((_K,), jnp.float32)
    z16 = jnp.zeros((_RPT, 16), jnp.float32)
    z1 = jnp.zeros((_RPT,), jnp.float32)

    part, partC, partS = _k1(src_r, dst_r, ea_p, x_p, cbuf, onesk, z16, z1)
    den = part[0, :n, 0:8] + part[1, :n, 0:8]
    num = part[0, :n, 8:16] + part[1, :n, 8:16]
    cnt = partC[0, :n] + partC[1, :n]
    ssum = partS[0, :n] + partS[1, :n]
    loop_attr = ssum / jnp.clip(cnt, 1.0)

    pre_l = xv[:, None] * (cs + cd) + loop_attr[:, None] * ce
    Ml = _leaky(B1 + xv[:, None] * cd)
    el1 = jnp.exp(_leaky(pre_l) - Ml)
    s = (num + el1 * xv[:, None]) / (den + el1 + 1e-16)
    out1 = (s[:, :, None] * w1[None]).reshape(n, _H1 * _C1) + b1
    h2 = jax.nn.elu(out1)
    g = h2 @ W2
    a2s = (g * a2_src[0, 0]).sum(-1)
    a2d = (g * a2_dst[0, 0]).sum(-1)
    ce2 = (We2[0] * ae2[0, 0]).sum()
    maxabs_eaf = jnp.maximum(maxabs_ea, jnp.max(jnp.abs(loop_attr)))
    B2 = jnp.max(jnp.abs(a2s)) + maxabs_eaf * jnp.abs(ce2)

    zpad = jnp.zeros((_NROW - _N,), jnp.float32)
    cbuf2 = jnp.stack([jnp.full((16,), ce2), jnp.full((16,), B2)])
    p2 = _k3(src_r, dst_r, ea_p,
             jnp.concatenate([a2s, zpad]),
             jnp.concatenate([g[:, 0], zpad]),
             jnp.concatenate([g[:, 1], zpad]),
             jnp.concatenate([g[:, 2], zpad]),
             jnp.concatenate([a2d, zpad]),
             cbuf2, z1)
    den2 = p2[0, 0, :n] + p2[1, 0, :n]
    num2 = (p2[0, 1:4, :n] + p2[1, 1:4, :n]).T

    pre2l = a2s + a2d + loop_attr * ce2
    M2l = _leaky(B2 + a2d)
    e2l = jnp.exp(_leaky(pre2l) - M2l)

    rows = 2000
    grid = (n // rows,)
    row_spec = pl.BlockSpec((rows, 3), lambda i: (i, 0))
    col_spec = pl.BlockSpec((rows, 1), lambda i: (i, 0))
    out2 = pl.pallas_call(
        _final_combine_kernel,
        grid=grid,
        in_specs=[row_spec, col_spec, row_spec, col_spec,
                  pl.BlockSpec((1, 3), lambda i: (0, 0))],
        out_specs=row_spec,
        out_shape=jax.ShapeDtypeStruct((n, 3), jnp.float32),
    )(num2, den2[:, None], g, e2l[:, None], b2[None, :])
    return out2
